# baseline (speedup 1.0000x reference)
import jax
import jax.numpy as jnp
from jax.experimental import pallas as pl

_ROWS_PER_BLOCK = 32
_N = 32768
_SPLIT = 96


def _masked_softmax_block(x_ref, m_ref, o_ref):
    x = x_ref[...]
    m = m_ref[...] == 1
    neg = jnp.finfo(x.dtype).min
    z = jnp.where(m, x, neg)
    mx = jnp.max(z, axis=-1, keepdims=True)
    e = jnp.exp(z - mx)
    s = jnp.sum(e, axis=-1, keepdims=True)
    scale = jnp.where(
        mx > neg,
        jnp.asarray(1.0, x.dtype) / jnp.maximum(s, jnp.asarray(1e-30, x.dtype)),
        jnp.zeros((), x.dtype),
    )
    o_ref[...] = e * scale


def _tc_part(x, m):
    rows, cols = x.shape
    rb = min(_ROWS_PER_BLOCK, rows)
    grid = (rows // rb,)
    spec = pl.BlockSpec((rb, cols), lambda i: (i, 0))
    return pl.pallas_call(
        _masked_softmax_block,
        grid=grid,
        in_specs=[spec, spec],
        out_specs=spec,
        out_shape=jax.ShapeDtypeStruct((rows, cols), x.dtype),
    )(x, m)


def kernel(inputLayer, mask):
    top = _tc_part(inputLayer[:_SPLIT], mask[:_SPLIT])
    bot = _tc_part(inputLayer[_SPLIT:], mask[_SPLIT:])
    return jnp.concatenate([top, bot], axis=0)


# speedup vs baseline: 1.8784x; 1.8784x over previous
import jax
import jax.numpy as jnp
from jax.experimental import pallas as pl

_ROWS_PER_BLOCK = 32
_N = 32768
_SPLIT = 96


def _masked_softmax_block(x_ref, m_ref, o_ref):
    x = x_ref[...]
    m = m_ref[...] == 1
    neg = jnp.finfo(x.dtype).min
    z = jnp.where(m, x, neg)
    mx = jnp.max(z, axis=-1, keepdims=True)
    e = jnp.exp(z - mx)
    s = jnp.sum(e, axis=-1, keepdims=True)
    scale = jnp.where(
        mx > neg,
        jnp.asarray(1.0, x.dtype) / jnp.maximum(s, jnp.asarray(1e-30, x.dtype)),
        jnp.zeros((), x.dtype),
    )
    o_ref[...] = e * scale


def _tc_part(x, m, row_start, n_rows):
    cols = x.shape[1]
    rb = min(_ROWS_PER_BLOCK, n_rows)
    base = row_start // rb
    in_spec = pl.BlockSpec((rb, cols), lambda i: (i + base, 0))
    out_spec = pl.BlockSpec((rb, cols), lambda i: (i, 0))
    return pl.pallas_call(
        _masked_softmax_block,
        grid=(n_rows // rb,),
        in_specs=[in_spec, in_spec],
        out_specs=out_spec,
        out_shape=jax.ShapeDtypeStruct((n_rows, cols), x.dtype),
    )(x, m)


def kernel(inputLayer, mask):
    rows = inputLayer.shape[0]
    top = _tc_part(inputLayer, mask, 0, _SPLIT)
    bot = _tc_part(inputLayer, mask, _SPLIT, rows - _SPLIT)
    return jnp.concatenate([top, bot], axis=0)
